# baseline (speedup 1.0000x reference)
import functools

import jax
import jax.numpy as jnp
from jax import lax
from jax.experimental import pallas as pl
from jax.experimental.pallas import tpu as pltpu
from jax.experimental.pallas import tpu_sc as plsc

_EPS = 1e-5
_BF = jnp.bfloat16


def _sc_gather(table, idx):
    V, W = table.shape
    B = idx.shape[0]
    NW = 32
    b_per_w = B // NW
    CH = next(c for c in range(128, 0, -8) if b_per_w % c == 0)
    n_ch = b_per_w // CH
    mesh = plsc.VectorSubcoreMesh(core_axis_name="c", subcore_axis_name="s")

    @functools.partial(
        pl.kernel, mesh=mesh,
        out_type=jax.ShapeDtypeStruct((B, W), table.dtype),
        scratch_types=[
            pltpu.VMEM((b_per_w,), jnp.int32),
            pltpu.VMEM((CH, W), table.dtype),
            pltpu.SemaphoreType.DMA,
        ],
    )
    def k(table_hbm, idx_hbm, out_hbm, idx_v, rows_v, sem):
        wid = lax.axis_index("s") * 2 + lax.axis_index("c")
        base = wid * b_per_w
        pltpu.sync_copy(idx_hbm.at[pl.ds(base, b_per_w)], idx_v)

        def body(c, carry):
            off = c * CH
            pltpu.async_copy(
                table_hbm.at[idx_v.at[pl.ds(off, CH)]], rows_v, sem).wait()
            pltpu.sync_copy(rows_v, out_hbm.at[pl.ds(base + off, CH)])
            return carry

        lax.fori_loop(0, n_ch, body, 0)

    return k(table, idx)


def _dot(x, w):
    return jnp.dot(x.astype(_BF), w, preferred_element_type=jnp.float32)


def _unpack(u):
    lo = lax.bitcast_convert_type(u << 16, jnp.float32)
    hi = lax.bitcast_convert_type(u & jnp.int32(-65536), jnp.float32)
    s = jnp.sum(lo, -1, keepdims=True) + jnp.sum(hi, -1, keepdims=True)
    q = (jnp.sum(lo * lo, -1, keepdims=True)
         + jnp.sum(hi * hi, -1, keepdims=True))
    x = jnp.concatenate([lo.astype(_BF), hi.astype(_BF)], axis=-1)
    return x, s, q


def _moments(sums, ssqs, d):
    mu = sums / d
    inv = lax.rsqrt(ssqs / d - mu * mu + _EPS)
    return mu * inv, inv


def _sum2(x):
    return jnp.sum(x, -1, keepdims=True), jnp.sum(x * x, -1, keepdims=True)


def _tc_body(ni_ref, nj_ref, e_ref, nd_ref,
             w0a, w0b, w0e, u0, v0,
             w1, u1, v1,
             wl, ul, vl,
             wtr1, ut1, vt1,
             wupa, wupb, u2, v2,
             wt2a, wt2b, u3, v3,
             wue, wut, u4, v4,
             nup_ref, eup_ref):
    ni, sum_ni, ssq_ni = _unpack(ni_ref[...])
    nj, sum_nj, ssq_nj = _unpack(nj_ref[...])
    e = e_ref[...]
    C = ni.shape[1]
    DE = e.shape[1]
    BE = ni.shape[0]
    BN = nup_ref.shape[0]

    sum_e, ssq_e = _sum2(e)

    mi0, inv0 = _moments(sum_ni + sum_nj + sum_e, ssq_ni + ssq_nj + ssq_e,
                         2 * C + DE)
    x = (_dot(ni, w0a[...]) + _dot(nj, w0b[...]) + _dot(e, w0e[...]))
    x = jax.nn.relu(x * inv0 - mi0 * u0[...] + v0[...])
    s, q = _sum2(x)
    mi1, inv1 = _moments(s, q, C)
    x = jax.nn.relu(_dot(x, w1[...]) * inv1 - mi1 * u1[...] + v1[...])
    s, q = _sum2(x)
    mil, invl = _moments(s, q, C)
    m = jax.nn.relu(_dot(x, wl[...]) * invl - mil * ul[...] + vl[...])
    m_i = jnp.sum(m.reshape(BN, BE // BN, C), axis=1)

    nd = nd_ref[...]
    s, q = _sum2(nd)
    min_, invn = _moments(s, q, C)
    h = jax.nn.relu(_dot(nd, wtr1[...]) * invn - min_ * ut1[...] + vt1[...])
    sh, qh = _sum2(h)
    sm, qm = _sum2(m_i)
    mi2, inv2 = _moments(sh + sm, qh + qm, 2 * C)
    nup = jax.nn.relu(
        (_dot(h, wupa[...]) + _dot(m_i, wupb[...])) * inv2
        - mi2 * u2[...] + v2[...])
    nup_ref[...] = nup

    mi3, inv3 = _moments(sum_ni + sum_nj, ssq_ni + ssq_nj, 2 * C)
    t = jax.nn.relu(
        (_dot(ni, wt2a[...]) + _dot(nj, wt2b[...])) * inv3
        - mi3 * u3[...] + v3[...])
    st, qt = _sum2(t)
    mi4, inv4 = _moments(sum_e + st, ssq_e + qt, DE + C)
    eup = jax.nn.relu(
        (_dot(e, wue[...]) + _dot(t, wut[...])) * inv4
        - mi4 * u4[...] + v4[...])
    eup_ref[...] = eup


def _fold(s, o, w, b):
    wp = w * s[:, None]
    u = jnp.sum(wp, axis=0).reshape(1, -1)
    v = (o @ w + b).reshape(1, -1)
    return wp, u, v


def kernel(nodes, edges, i, j, params):
    N, C = nodes.shape
    E, DE = edges.shape
    H = C // 2
    P = params

    idx = jnp.concatenate([i, j]).astype(jnp.int32)
    nb = nodes.astype(_BF)
    table = lax.bitcast_convert_type(
        jnp.stack([nb[:, :H], nb[:, H:]], axis=-1), jnp.int32)
    gath = _sc_gather(table, idx)

    bf = lambda w: w.astype(_BF)
    w0p, u0, v0 = _fold(P["msg_ln0"]["s"], P["msg_ln0"]["o"],
                        P["msg_l0"]["w"], P["msg_l0"]["b"])
    w1p, u1, v1 = _fold(P["msg_ln1"]["s"], P["msg_ln1"]["o"],
                        P["msg_l1"]["w"], P["msg_l1"]["b"])
    wlp, ul, vl = _fold(P["msg_lnl"]["s"], P["msg_lnl"]["o"],
                        P["msg_ll"]["w"], P["msg_ll"]["b"])
    wt1p, ut1, vt1 = _fold(P["ln1"]["s"], P["ln1"]["o"],
                           P["tr1"]["w"], P["tr1"]["b"])
    wupp, u2, v2 = _fold(P["ln2"]["s"], P["ln2"]["o"],
                         P["up"]["w"], P["up"]["b"])
    wt2p, u3, v3 = _fold(P["ln3"]["s"], P["ln3"]["o"],
                         P["tr2"]["w"], P["tr2"]["b"])
    wuep, u4, v4 = _fold(P["ln4"]["s"], P["ln4"]["o"],
                         P["eup"]["w"], P["eup"]["b"])
    plist = [
        bf(w0p[:C]), bf(w0p[C:2 * C]), bf(w0p[2 * C:]), u0, v0,
        bf(w1p), u1, v1,
        bf(wlp), ul, vl,
        bf(wt1p), ut1, vt1,
        bf(wupp[:C]), bf(wupp[C:]), u2, v2,
        bf(wt2p[:C]), bf(wt2p[C:]), u3, v3,
        bf(wuep[:DE]), bf(wuep[DE:]), u4, v4,
    ]

    BE = 3200
    BN = BE // (E // N)
    K = 10
    SE = E // K
    Gk = SE // BE
    full = lambda p: pl.BlockSpec(p.shape, lambda b: tuple(0 for _ in p.shape))
    hbm = pl.BlockSpec(memory_space=pltpu.MemorySpace.HBM)
    n_main = 4 + len(plist)

    def chunk_body(*refs):
        _tc_body(*(refs[:n_main] + refs[n_main + 2:]))

    gaths = []
    for k in range(K):
        idx_k = jnp.concatenate(
            [i[k * SE:(k + 1) * SE], j[k * SE:(k + 1) * SE]]).astype(jnp.int32)
        gaths.append(_sc_gather(table, idx_k))

    n_up = e_up = None
    for k in range(K):
        base = k * Gk
        in_specs = [
            pl.BlockSpec((BE, H), lambda b: (b, 0)),
            pl.BlockSpec((BE, H), lambda b: (b + Gk, 0)),
            pl.BlockSpec((BE, DE), lambda b, base=base: (base + b, 0)),
            pl.BlockSpec((BN, C), lambda b, base=base: (base + b, 0)),
        ] + [full(p) for p in plist]
        out_specs = [
            pl.BlockSpec((BN, C), lambda b, base=base: (base + b, 0)),
            pl.BlockSpec((BE, C), lambda b, base=base: (base + b, 0)),
        ]
        args = [gaths[k], gaths[k], edges, nodes] + plist
        kw = {}
        if k == 0:
            body = _tc_body
        else:
            body = chunk_body
            in_specs += [hbm, hbm]
            args += [n_up, e_up]
            kw["input_output_aliases"] = {n_main: 0, n_main + 1: 1}
        n_up, e_up = pl.pallas_call(
            body,
            grid=(Gk,),
            in_specs=in_specs,
            out_specs=out_specs,
            out_shape=[
                jax.ShapeDtypeStruct((N, C), jnp.float32),
                jax.ShapeDtypeStruct((E, C), jnp.float32),
            ],
            **kw,
        )(*args)
    return (n_up, e_up)


# speedup vs baseline: 1.1121x; 1.1121x over previous
import functools

import jax
import jax.numpy as jnp
from jax import lax
from jax.experimental import pallas as pl
from jax.experimental.pallas import tpu as pltpu
from jax.experimental.pallas import tpu_sc as plsc

_EPS = 1e-5
_BF = jnp.bfloat16


def _sc_gather(table, idx):
    V, W = table.shape
    B = idx.shape[0]
    NW = 32
    b_per_w = B // NW
    CH = next(c for c in range(128, 0, -8) if b_per_w % c == 0)
    n_ch = b_per_w // CH
    mesh = plsc.VectorSubcoreMesh(core_axis_name="c", subcore_axis_name="s")

    @functools.partial(
        pl.kernel, mesh=mesh,
        out_type=jax.ShapeDtypeStruct((B, W), table.dtype),
        scratch_types=[
            pltpu.VMEM((b_per_w,), jnp.int32),
            pltpu.VMEM((CH, W), table.dtype),
            pltpu.SemaphoreType.DMA,
        ],
    )
    def k(table_hbm, idx_hbm, out_hbm, idx_v, rows_v, sem):
        wid = lax.axis_index("s") * 2 + lax.axis_index("c")
        base = wid * b_per_w
        pltpu.sync_copy(idx_hbm.at[pl.ds(base, b_per_w)], idx_v)

        def body(c, carry):
            off = c * CH
            pltpu.async_copy(
                table_hbm.at[idx_v.at[pl.ds(off, CH)]], rows_v, sem).wait()
            pltpu.sync_copy(rows_v, out_hbm.at[pl.ds(base + off, CH)])
            return carry

        lax.fori_loop(0, n_ch, body, 0)

    return k(table, idx)


def _dot(x, w):
    return jnp.dot(x.astype(_BF), w, preferred_element_type=jnp.float32)


def _unpack(u):
    lo = lax.bitcast_convert_type(u << 16, jnp.float32)
    hi = lax.bitcast_convert_type(u & jnp.int32(-65536), jnp.float32)
    x = jnp.concatenate([lo, hi], axis=-1)
    s, q = _sum2(x)
    return x, s, q


def _sum2(x):
    return jnp.sum(x, -1, keepdims=True), jnp.sum(x * x, -1, keepdims=True)


def _mom(s_, q_, d):
    mu = s_ / d
    inv = lax.rsqrt(q_ / d - mu * mu + _EPS)
    return mu * inv, inv


def _epi(z, mi, inv, uv):
    return jax.nn.relu(
        z * inv + jnp.dot(mi, uv[0:1], preferred_element_type=jnp.float32)
        + uv[1:2])


def _tc_body(ni_ref, nj_ref, e_ref, nd_ref,
             w0a, w0b, w0e, uv0,
             w1, uv1,
             wl, uvl,
             wtr1, uvt1,
             wupa, wupb, uv2,
             wt2a, wt2b, uv3,
             wue, wut, uv4,
             nup_ref, eup_ref):
    ni, sum_ni, ssq_ni = _unpack(ni_ref[...])
    nj, sum_nj, ssq_nj = _unpack(nj_ref[...])
    e = e_ref[...]
    C = ni.shape[1]
    DE = e.shape[1]
    BE = ni.shape[0]
    BN = nup_ref.shape[0]

    sum_e, ssq_e = _sum2(e)

    mi0, inv0 = _mom(sum_ni + sum_nj + sum_e, ssq_ni + ssq_nj + ssq_e,
                     2 * C + DE)
    x = _epi(_dot(ni, w0a[...]) + _dot(nj, w0b[...]) + _dot(e, w0e[...]),
             mi0, inv0, uv0[...])
    s, q = _sum2(x)
    mi1, inv1 = _mom(s, q, C)
    x = _epi(_dot(x, w1[...]), mi1, inv1, uv1[...])
    s, q = _sum2(x)
    mil, invl = _mom(s, q, C)
    m = _epi(_dot(x, wl[...]), mil, invl, uvl[...])
    m_i = jnp.sum(m.reshape(BN, BE // BN, C), axis=1)

    nd = nd_ref[...]
    s, q = _sum2(nd)
    min_, invn = _mom(s, q, C)
    h = _epi(_dot(nd, wtr1[...]), min_, invn, uvt1[...])
    sh, qh = _sum2(h)
    sm, qm = _sum2(m_i)
    mi2, inv2 = _mom(sh + sm, qh + qm, 2 * C)
    nup_ref[...] = _epi(_dot(h, wupa[...]) + _dot(m_i, wupb[...]),
                        mi2, inv2, uv2[...])

    mi3, inv3 = _mom(sum_ni + sum_nj, ssq_ni + ssq_nj, 2 * C)
    t = _epi(_dot(ni, wt2a[...]) + _dot(nj, wt2b[...]), mi3, inv3, uv3[...])
    st, qt = _sum2(t)
    mi4, inv4 = _mom(sum_e + st, ssq_e + qt, DE + C)
    eup_ref[...] = _epi(_dot(e, wue[...]) + _dot(t, wut[...]),
                        mi4, inv4, uv4[...])


def _fold(s, o, w, b):
    wp = w * s[:, None]
    u = jnp.sum(wp, axis=0)
    v = o @ w + b
    return wp, jnp.stack([-u, v])


def kernel(nodes, edges, i, j, params):
    N, C = nodes.shape
    E, DE = edges.shape
    H = C // 2
    P = params

    nb = nodes.astype(_BF)
    table = lax.bitcast_convert_type(
        jnp.stack([nb[:, :H], nb[:, H:]], axis=-1), jnp.int32)

    bf = lambda w: w.astype(_BF)
    w0p, uv0 = _fold(P["msg_ln0"]["s"], P["msg_ln0"]["o"],
                     P["msg_l0"]["w"], P["msg_l0"]["b"])
    w1p, uv1 = _fold(P["msg_ln1"]["s"], P["msg_ln1"]["o"],
                     P["msg_l1"]["w"], P["msg_l1"]["b"])
    wlp, uvl = _fold(P["msg_lnl"]["s"], P["msg_lnl"]["o"],
                     P["msg_ll"]["w"], P["msg_ll"]["b"])
    wt1p, uvt1 = _fold(P["ln1"]["s"], P["ln1"]["o"],
                       P["tr1"]["w"], P["tr1"]["b"])
    wupp, uv2 = _fold(P["ln2"]["s"], P["ln2"]["o"],
                      P["up"]["w"], P["up"]["b"])
    wt2p, uv3 = _fold(P["ln3"]["s"], P["ln3"]["o"],
                      P["tr2"]["w"], P["tr2"]["b"])
    wuep, uv4 = _fold(P["ln4"]["s"], P["ln4"]["o"],
                      P["eup"]["w"], P["eup"]["b"])
    plist = [
        bf(w0p[:C]), bf(w0p[C:2 * C]), bf(w0p[2 * C:]), uv0,
        bf(w1p), uv1,
        bf(wlp), uvl,
        bf(wt1p), uvt1,
        bf(wupp[:C]), bf(wupp[C:]), uv2,
        bf(wt2p[:C]), bf(wt2p[C:]), uv3,
        bf(wuep[:DE]), bf(wuep[DE:]), uv4,
    ]

    BE = 3200
    BN = BE // (E // N)
    K = 5
    SE = E // K
    Gk = SE // BE
    full = lambda p: pl.BlockSpec(p.shape, lambda b: tuple(0 for _ in p.shape))
    hbm = pl.BlockSpec(memory_space=pltpu.MemorySpace.HBM)
    n_main = 4 + len(plist)

    def chunk_body(*refs):
        _tc_body(*(refs[:n_main] + refs[n_main + 2:]))

    gaths = []
    for k in range(K):
        idx_k = jnp.concatenate(
            [i[k * SE:(k + 1) * SE], j[k * SE:(k + 1) * SE]]).astype(jnp.int32)
        gaths.append(_sc_gather(table, idx_k))

    n_up = e_up = None
    for k in range(K):
        base = k * Gk
        in_specs = [
            pl.BlockSpec((BE, H), lambda b: (b, 0)),
            pl.BlockSpec((BE, H), lambda b: (b + Gk, 0)),
            pl.BlockSpec((BE, DE), lambda b, base=base: (base + b, 0)),
            pl.BlockSpec((BN, C), lambda b, base=base: (base + b, 0)),
        ] + [full(p) for p in plist]
        out_specs = [
            pl.BlockSpec((BN, C), lambda b, base=base: (base + b, 0)),
            pl.BlockSpec((BE, C), lambda b, base=base: (base + b, 0)),
        ]
        args = [gaths[k], gaths[k], edges, nodes] + plist
        kw = {}
        if k == 0:
            body = _tc_body
        else:
            body = chunk_body
            in_specs += [hbm, hbm]
            args += [n_up, e_up]
            kw["input_output_aliases"] = {n_main: 0, n_main + 1: 1}
        n_up, e_up = pl.pallas_call(
            body,
            grid=(Gk,),
            in_specs=in_specs,
            out_specs=out_specs,
            out_shape=[
                jax.ShapeDtypeStruct((N, C), jnp.float32),
                jax.ShapeDtypeStruct((E, C), jnp.float32),
            ],
            **kw,
        )(*args)
    return (n_up, e_up)
